# Initial kernel scaffold; baseline (speedup 1.0000x reference)
import jax
import jax.numpy as jnp
from jax.experimental import pallas as pl


def kernel(xyz, points, W0, b0, g0, beta0, W1, b1, g1, beta1, W2, b2, g2, beta2):
    raise NotImplementedError("write your pallas kernel here")


# speedup vs baseline: 4.9921x; 4.9921x over previous
import functools

import jax
import jax.numpy as jnp
from jax import lax
from jax.experimental import pallas as pl
from jax.experimental.pallas import tpu as pltpu
from jax.experimental.pallas import tpu_sc as plsc

KNN = 32
NQ = 128
QB = 64
EPS = 1e-5


def _knn_body(xyzt_ref, xyzq_ref, ptsq_ref, w0t_ref, b0_ref, idx_ref, v_ref, r_ref):
    b = pl.program_id(0)
    N = xyzt_ref.shape[2]
    xt = xyzt_ref[0]
    q = xyzq_ref[0]
    X0, X1, X2 = xt[0:1, :], xt[1:2, :], xt[2:3, :]
    x0, x1, x2 = q[:, 0:1], q[:, 1:2], q[:, 2:3]
    sqm = X0 * X0 + X1 * X1 + X2 * X2
    sqn = x0 * x0 + x1 * x1 + x2 * x2
    dot = lax.dot_general(q, xt, (((1,), (0,)), ((), ())),
                          preferred_element_type=jnp.float32)
    S = (sqn + sqm) - 2.0 * dot

    col = lax.broadcasted_iota(jnp.int32, (NQ, N), 1)
    kcol = lax.broadcasted_iota(jnp.int32, (NQ, KNN), 1)
    BIG = jnp.float32(jnp.finfo(jnp.float32).max)

    def body(i, carry):
        s, ids = carry
        m = jnp.min(s, axis=1, keepdims=True)
        am = jnp.min(jnp.where(s == m, col, N), axis=1, keepdims=True)
        ids = jnp.where(kcol == i, am, ids)
        s = jnp.where(col == am, BIG, s)
        return s, ids

    ids0 = jnp.full((NQ, KNN), 0, jnp.int32)
    _, ids = lax.fori_loop(0, KNN, body, (S, ids0))
    idx_ref[0] = ids + b * N

    w0t = w0t_ref[...]
    r = (x0 * w0t[0:1, :] + x1 * w0t[1:2, :] + x2 * w0t[2:3, :])
    pts = ptsq_ref[0]
    v = jnp.dot(pts, w0t[3:67, :], preferred_element_type=jnp.float32,
                precision=lax.Precision.HIGHEST)
    v_ref[0] = v + r + b0_ref[...]
    r_ref[0] = r


def _knn_call(xyzt, xyz, points, W0T, b0r):
    B, N, _ = xyz.shape
    grid = (B, N // NQ)
    return pl.pallas_call(
        _knn_body,
        grid=grid,
        in_specs=[
            pl.BlockSpec((1, 3, N), lambda b, q: (b, 0, 0)),
            pl.BlockSpec((1, NQ, 3), lambda b, q: (b, q, 0)),
            pl.BlockSpec((1, NQ, 64), lambda b, q: (b, q, 0)),
            pl.BlockSpec((67, 64), lambda b, q: (0, 0)),
            pl.BlockSpec((1, 64), lambda b, q: (0, 0)),
        ],
        out_specs=[
            pl.BlockSpec((1, NQ, KNN), lambda b, q: (b, q, 0)),
            pl.BlockSpec((1, NQ, 64), lambda b, q: (b, q, 0)),
            pl.BlockSpec((1, NQ, 64), lambda b, q: (b, q, 0)),
        ],
        out_shape=[
            jax.ShapeDtypeStruct((B, N, KNN), jnp.int32),
            jax.ShapeDtypeStruct((B, N, 64), jnp.float32),
            jax.ShapeDtypeStruct((B, N, 64), jnp.float32),
        ],
    )(xyzt, xyz, points, W0T, b0r)


def _sc_gather(table, idx3):
    NW = 32
    NCHUNK = idx3.shape[1]
    mesh = plsc.VectorSubcoreMesh(core_axis_name="c", subcore_axis_name="s")

    @functools.partial(
        pl.kernel,
        mesh=mesh,
        compiler_params=pltpu.CompilerParams(use_tc_tiling_on_sc=False),
        out_type=jax.ShapeDtypeStruct((NW * NCHUNK, 128, 64), jnp.float32),
        scratch_types=[
            pltpu.VMEM((NCHUNK, 128), jnp.int32),
            pltpu.VMEM((128, 64), jnp.float32),
            pltpu.SemaphoreType.DMA,
        ],
    )
    def k(table_hbm, idx_hbm, out_hbm, idx_v, rows_v, sem):
        w = lax.axis_index("s") * 2 + lax.axis_index("c")
        pltpu.sync_copy(idx_hbm.at[w], idx_v)

        def body(j, carry):
            pltpu.async_copy(table_hbm.at[idx_v.at[j]], rows_v, sem).wait()
            pltpu.sync_copy(rows_v, out_hbm.at[w * NCHUNK + j])
            return carry

        lax.fori_loop(0, NCHUNK, body, 0)

    return k(table, idx3)


def _stats_body(raw_ref, r_ref, out_ref):
    g = pl.program_id(0)
    raw = raw_ref[...]
    r = r_ref[...]
    x1 = raw - jnp.repeat(r, KNN, axis=0)
    s = jnp.sum(x1, axis=0, keepdims=True)
    s2 = jnp.sum(x1 * x1, axis=0, keepdims=True)

    @pl.when(g == 0)
    def _():
        out_ref[...] = jnp.zeros_like(out_ref)

    out_ref[0:1, :] += s
    out_ref[1:2, :] += s2


def _stats_call(raw, R):
    BN = R.shape[0]
    grid = (BN // QB,)
    return pl.pallas_call(
        _stats_body,
        grid=grid,
        in_specs=[
            pl.BlockSpec((QB * KNN, 64), lambda g: (g, 0)),
            pl.BlockSpec((QB, 64), lambda g: (g, 0)),
        ],
        out_specs=pl.BlockSpec((8, 64), lambda g: (0, 0)),
        out_shape=jax.ShapeDtypeStruct((8, 64), jnp.float32),
    )(raw, R)


def _layer_body(raw_ref, r_ref, sc_ref, sh_ref, wt_ref, x2_ref, st_ref):
    g = pl.program_id(0)
    raw = raw_ref[...]
    r = r_ref[...]
    if r.shape[0] == QB:
        x = raw - jnp.repeat(r, KNN, axis=0)
    else:
        x = raw
    h = jnp.maximum(x * sc_ref[...] + sh_ref[...], 0.0)
    x2 = jnp.dot(h, wt_ref[...], preferred_element_type=jnp.float32,
                 precision=lax.Precision.HIGHEST)
    x2_ref[...] = x2
    s = jnp.sum(x2, axis=0, keepdims=True)
    s2 = jnp.sum(x2 * x2, axis=0, keepdims=True)

    @pl.when(g == 0)
    def _():
        st_ref[...] = jnp.zeros_like(st_ref)

    st_ref[0:1, :] += s
    st_ref[1:2, :] += s2


def _layer_call(raw, R, scale, shift, WT):
    M, Cin = raw.shape
    Cout = WT.shape[1]
    grid = (M // (QB * KNN),)
    use_r = R is not None
    rr = R if use_r else jnp.zeros((1, 1), jnp.float32)
    rspec = (pl.BlockSpec((QB, Cin), lambda g: (g, 0)) if use_r
             else pl.BlockSpec((1, 1), lambda g: (0, 0)))
    return pl.pallas_call(
        _layer_body,
        grid=grid,
        in_specs=[
            pl.BlockSpec((QB * KNN, Cin), lambda g: (g, 0)),
            rspec,
            pl.BlockSpec((1, Cin), lambda g: (0, 0)),
            pl.BlockSpec((1, Cin), lambda g: (0, 0)),
            pl.BlockSpec((Cin, Cout), lambda g: (0, 0)),
        ],
        out_specs=[
            pl.BlockSpec((QB * KNN, Cout), lambda g: (g, 0)),
            pl.BlockSpec((8, Cout), lambda g: (0, 0)),
        ],
        out_shape=[
            jax.ShapeDtypeStruct((M, Cout), jnp.float32),
            jax.ShapeDtypeStruct((8, Cout), jnp.float32),
        ],
    )(raw, rr, scale, shift, WT)


def _final_body(x3_ref, sc_ref, sh_ref, out_ref):
    x3 = x3_ref[...]
    y = jnp.maximum(x3 * sc_ref[...] + sh_ref[...], 0.0)
    y = y.reshape(QB, KNN, 128)
    out_ref[...] = jnp.max(y, axis=1)


def _final_call(x3, scale, shift):
    M = x3.shape[0]
    BN = M // KNN
    grid = (BN // QB,)
    return pl.pallas_call(
        _final_body,
        grid=grid,
        in_specs=[
            pl.BlockSpec((QB * KNN, 128), lambda g: (g, 0)),
            pl.BlockSpec((1, 128), lambda g: (0, 0)),
            pl.BlockSpec((1, 128), lambda g: (0, 0)),
        ],
        out_specs=pl.BlockSpec((QB, 128), lambda g: (g, 0)),
        out_shape=jax.ShapeDtypeStruct((BN, 128), jnp.float32),
    )(x3, scale, shift)


def _finalize(stats, cnt, gm, bt):
    s = stats[0, :gm.shape[0]]
    s2 = stats[1, :gm.shape[0]]
    mu = s / cnt
    var = s2 / cnt - mu * mu
    scale = gm / jnp.sqrt(var + EPS)
    shift = bt - mu * scale
    return scale[None, :], shift[None, :]


def kernel(xyz, points, W0, b0, g0, beta0, W1, b1, g1, beta1, W2, b2, g2, beta2):
    B, N, _ = xyz.shape
    BN = B * N
    cnt = jnp.float32(BN * KNN)

    xyzt = jnp.swapaxes(xyz, 1, 2)
    W0T = jnp.swapaxes(W0, 0, 1)
    idx, V, R = _knn_call(xyzt, xyz, points, W0T, b0[None, :])

    idx_flat = idx.reshape(-1)
    NW = 32
    NCHUNK = (BN * KNN) // (NW * 128)
    idx3 = idx_flat.reshape(NW, NCHUNK, 128)
    raw = _sc_gather(V.reshape(BN, 64), idx3).reshape(BN * KNN, 64)

    Rf = R.reshape(BN, 64)
    st1 = _stats_call(raw, Rf)
    sc1, sh1 = _finalize(st1, cnt, g0, beta0)

    W1T = jnp.swapaxes(W1, 0, 1)
    x2, st2 = _layer_call(raw, Rf, sc1, sh1, W1T)
    sc2, sh2 = _finalize(st2, cnt, g1, beta1)

    W2T = jnp.swapaxes(W2, 0, 1)
    x3, st3 = _layer_call(x2, None, sc2, sh2, W2T)
    sc3, sh3 = _finalize(st3, cnt, g2, beta2)

    out = _final_call(x3, sc3, sh3)
    return out.reshape(B, N, 128)


# speedup vs baseline: 5.3820x; 1.0781x over previous
import functools

import jax
import jax.numpy as jnp
from jax import lax
from jax.experimental import pallas as pl
from jax.experimental.pallas import tpu as pltpu
from jax.experimental.pallas import tpu_sc as plsc

KNN = 32
NQ = 256
QB = 128
EPS = 1e-5


def _knn_body(xyzt_ref, xyzq_ref, ptsq_ref, w0t_ref, b0_ref, idx_ref, v_ref, r_ref):
    b = pl.program_id(0)
    N = xyzt_ref.shape[2]
    xt = xyzt_ref[0]
    q = xyzq_ref[0]
    X0, X1, X2 = xt[0:1, :], xt[1:2, :], xt[2:3, :]
    x0, x1, x2 = q[:, 0:1], q[:, 1:2], q[:, 2:3]
    sqm = X0 * X0 + X1 * X1 + X2 * X2
    sqn = x0 * x0 + x1 * x1 + x2 * x2
    dot = lax.dot_general(q, xt, (((1,), (0,)), ((), ())),
                          preferred_element_type=jnp.float32)
    S = (sqn + sqm) - 2.0 * dot

    col = lax.broadcasted_iota(jnp.int32, (NQ, N), 1)
    kcol = lax.broadcasted_iota(jnp.int32, (NQ, KNN), 1)
    BIG = jnp.float32(jnp.finfo(jnp.float32).max)

    def body(i, carry):
        s, ids = carry
        m = jnp.min(s, axis=1, keepdims=True)
        am = jnp.min(jnp.where(s == m, col, N), axis=1, keepdims=True)
        ids = jnp.where(kcol == i, am, ids)
        s = jnp.where(col == am, BIG, s)
        return s, ids

    ids0 = jnp.full((NQ, KNN), 0, jnp.int32)
    _, ids = lax.fori_loop(0, KNN, body, (S, ids0))
    idx_ref[0] = ids + b * N

    w0t = w0t_ref[...]
    r = (x0 * w0t[0:1, :] + x1 * w0t[1:2, :] + x2 * w0t[2:3, :])
    pts = ptsq_ref[0]
    v = jnp.dot(pts, w0t[3:67, :], preferred_element_type=jnp.float32,
                precision=lax.Precision.HIGHEST)
    v_ref[0] = v + r + b0_ref[...]
    r_ref[0] = r


def _knn_call(xyzt, xyz, points, W0T, b0r):
    B, N, _ = xyz.shape
    grid = (B, N // NQ)
    return pl.pallas_call(
        _knn_body,
        grid=grid,
        in_specs=[
            pl.BlockSpec((1, 3, N), lambda b, q: (b, 0, 0)),
            pl.BlockSpec((1, NQ, 3), lambda b, q: (b, q, 0)),
            pl.BlockSpec((1, NQ, 64), lambda b, q: (b, q, 0)),
            pl.BlockSpec((67, 64), lambda b, q: (0, 0)),
            pl.BlockSpec((1, 64), lambda b, q: (0, 0)),
        ],
        out_specs=[
            pl.BlockSpec((1, NQ, KNN), lambda b, q: (b, q, 0)),
            pl.BlockSpec((1, NQ, 64), lambda b, q: (b, q, 0)),
            pl.BlockSpec((1, NQ, 64), lambda b, q: (b, q, 0)),
        ],
        out_shape=[
            jax.ShapeDtypeStruct((B, N, KNN), jnp.int32),
            jax.ShapeDtypeStruct((B, N, 64), jnp.float32),
            jax.ShapeDtypeStruct((B, N, 64), jnp.float32),
        ],
    )(xyzt, xyz, points, W0T, b0r)


def _sc_gather(table, idx3):
    NW = 32
    NCHUNK = idx3.shape[1]
    mesh = plsc.VectorSubcoreMesh(core_axis_name="c", subcore_axis_name="s")

    @functools.partial(
        pl.kernel,
        mesh=mesh,
        compiler_params=pltpu.CompilerParams(use_tc_tiling_on_sc=False),
        out_type=jax.ShapeDtypeStruct((NW * NCHUNK, 128, 64), jnp.float32),
        scratch_types=[
            pltpu.VMEM((NCHUNK, 128), jnp.int32),
            pltpu.VMEM((128, 64), jnp.float32),
            pltpu.SemaphoreType.DMA,
        ],
    )
    def k(table_hbm, idx_hbm, out_hbm, idx_v, rows_v, sem):
        w = lax.axis_index("s") * 2 + lax.axis_index("c")
        pltpu.sync_copy(idx_hbm.at[w], idx_v)

        def body(j, carry):
            pltpu.async_copy(table_hbm.at[idx_v.at[j]], rows_v, sem).wait()
            pltpu.sync_copy(rows_v, out_hbm.at[w * NCHUNK + j])
            return carry

        lax.fori_loop(0, NCHUNK, body, 0)

    return k(table, idx3)


def _stats_body(raw_ref, r_ref, out_ref):
    g = pl.program_id(0)
    raw = raw_ref[...]
    r = r_ref[...]
    x1 = raw - jnp.repeat(r, KNN, axis=0)
    s = jnp.sum(x1, axis=0, keepdims=True)
    s2 = jnp.sum(x1 * x1, axis=0, keepdims=True)

    @pl.when(g == 0)
    def _():
        out_ref[...] = jnp.zeros_like(out_ref)

    out_ref[0:1, :] += s
    out_ref[1:2, :] += s2


def _stats_call(raw, R):
    BN = R.shape[0]
    grid = (BN // QB,)
    return pl.pallas_call(
        _stats_body,
        grid=grid,
        in_specs=[
            pl.BlockSpec((QB * KNN, 64), lambda g: (g, 0)),
            pl.BlockSpec((QB, 64), lambda g: (g, 0)),
        ],
        out_specs=pl.BlockSpec((8, 64), lambda g: (0, 0)),
        out_shape=jax.ShapeDtypeStruct((8, 64), jnp.float32),
    )(raw, R)


def _layer_body(raw_ref, r_ref, sc_ref, sh_ref, wt_ref, x2_ref, st_ref):
    g = pl.program_id(0)
    raw = raw_ref[...]
    r = r_ref[...]
    if r.shape[0] == QB:
        x = raw - jnp.repeat(r, KNN, axis=0)
    else:
        x = raw
    h = jnp.maximum(x * sc_ref[...] + sh_ref[...], 0.0)
    x2 = jnp.dot(h, wt_ref[...], preferred_element_type=jnp.float32,
                 precision=lax.Precision.HIGHEST)
    x2_ref[...] = x2
    s = jnp.sum(x2, axis=0, keepdims=True)
    s2 = jnp.sum(x2 * x2, axis=0, keepdims=True)

    @pl.when(g == 0)
    def _():
        st_ref[...] = jnp.zeros_like(st_ref)

    st_ref[0:1, :] += s
    st_ref[1:2, :] += s2


def _layer_call(raw, R, scale, shift, WT):
    M, Cin = raw.shape
    Cout = WT.shape[1]
    grid = (M // (QB * KNN),)
    use_r = R is not None
    rr = R if use_r else jnp.zeros((1, 1), jnp.float32)
    rspec = (pl.BlockSpec((QB, Cin), lambda g: (g, 0)) if use_r
             else pl.BlockSpec((1, 1), lambda g: (0, 0)))
    return pl.pallas_call(
        _layer_body,
        grid=grid,
        in_specs=[
            pl.BlockSpec((QB * KNN, Cin), lambda g: (g, 0)),
            rspec,
            pl.BlockSpec((1, Cin), lambda g: (0, 0)),
            pl.BlockSpec((1, Cin), lambda g: (0, 0)),
            pl.BlockSpec((Cin, Cout), lambda g: (0, 0)),
        ],
        out_specs=[
            pl.BlockSpec((QB * KNN, Cout), lambda g: (g, 0)),
            pl.BlockSpec((8, Cout), lambda g: (0, 0)),
        ],
        out_shape=[
            jax.ShapeDtypeStruct((M, Cout), jnp.float32),
            jax.ShapeDtypeStruct((8, Cout), jnp.float32),
        ],
    )(raw, rr, scale, shift, WT)


def _final_body(x3_ref, sc_ref, sh_ref, out_ref):
    x3 = x3_ref[...]
    y = jnp.maximum(x3 * sc_ref[...] + sh_ref[...], 0.0)
    y = y.reshape(QB, KNN, 128)
    out_ref[...] = jnp.max(y, axis=1)


def _final_call(x3, scale, shift):
    M = x3.shape[0]
    BN = M // KNN
    grid = (BN // QB,)
    return pl.pallas_call(
        _final_body,
        grid=grid,
        in_specs=[
            pl.BlockSpec((QB * KNN, 128), lambda g: (g, 0)),
            pl.BlockSpec((1, 128), lambda g: (0, 0)),
            pl.BlockSpec((1, 128), lambda g: (0, 0)),
        ],
        out_specs=pl.BlockSpec((QB, 128), lambda g: (g, 0)),
        out_shape=jax.ShapeDtypeStruct((BN, 128), jnp.float32),
    )(x3, scale, shift)


def _finalize(stats, cnt, gm, bt):
    s = stats[0, :gm.shape[0]]
    s2 = stats[1, :gm.shape[0]]
    mu = s / cnt
    var = s2 / cnt - mu * mu
    scale = gm / jnp.sqrt(var + EPS)
    shift = bt - mu * scale
    return scale[None, :], shift[None, :]


def kernel(xyz, points, W0, b0, g0, beta0, W1, b1, g1, beta1, W2, b2, g2, beta2):
    B, N, _ = xyz.shape
    BN = B * N
    cnt = jnp.float32(BN * KNN)

    xyzt = jnp.swapaxes(xyz, 1, 2)
    W0T = jnp.swapaxes(W0, 0, 1)
    idx, V, R = _knn_call(xyzt, xyz, points, W0T, b0[None, :])

    idx_flat = idx.reshape(-1)
    NW = 32
    NCHUNK = (BN * KNN) // (NW * 128)
    idx3 = idx_flat.reshape(NW, NCHUNK, 128)
    raw = _sc_gather(V.reshape(BN, 64), idx3).reshape(BN * KNN, 64)

    Rf = R.reshape(BN, 64)
    st1 = _stats_call(raw, Rf)
    sc1, sh1 = _finalize(st1, cnt, g0, beta0)

    W1T = jnp.swapaxes(W1, 0, 1)
    x2, st2 = _layer_call(raw, Rf, sc1, sh1, W1T)
    sc2, sh2 = _finalize(st2, cnt, g1, beta1)

    W2T = jnp.swapaxes(W2, 0, 1)
    x3, st3 = _layer_call(x2, None, sc2, sh2, W2T)
    sc3, sh3 = _finalize(st3, cnt, g2, beta2)

    out = _final_call(x3, sc3, sh3)
    return out.reshape(B, N, 128)


# speedup vs baseline: 5.4779x; 1.0178x over previous
import functools

import jax
import jax.numpy as jnp
from jax import lax
from jax.experimental import pallas as pl
from jax.experimental.pallas import tpu as pltpu
from jax.experimental.pallas import tpu_sc as plsc

KNN = 32
NQ = 256
QB = 128
EPS = 1e-5


def _knn_body(xyzt_ref, xyzq_ref, ptsq_ref, w0t_ref, b0_ref, idx_ref, v_ref, r_ref):
    b = pl.program_id(0)
    N = xyzt_ref.shape[2]
    xt = xyzt_ref[0]
    q = xyzq_ref[0]
    X0, X1, X2 = xt[0:1, :], xt[1:2, :], xt[2:3, :]
    x0, x1, x2 = q[:, 0:1], q[:, 1:2], q[:, 2:3]
    sqm = X0 * X0 + X1 * X1 + X2 * X2
    sqn = x0 * x0 + x1 * x1 + x2 * x2
    dot = lax.dot_general(q, xt, (((1,), (0,)), ((), ())),
                          preferred_element_type=jnp.float32)
    S = (sqn + sqm) - 2.0 * dot

    col = lax.broadcasted_iota(jnp.int32, (NQ, N), 1)
    kcol = lax.broadcasted_iota(jnp.int32, (NQ, KNN), 1)
    BIG = jnp.float32(jnp.finfo(jnp.float32).max)

    def body(i, carry):
        s, ids = carry
        am = jnp.argmin(s, axis=1).astype(jnp.int32)[:, None]
        ids = jnp.where(kcol == i, am, ids)
        s = jnp.where(col == am, BIG, s)
        return s, ids

    ids0 = jnp.full((NQ, KNN), 0, jnp.int32)
    _, ids = lax.fori_loop(0, KNN, body, (S, ids0))
    idx_ref[0] = ids + b * N

    w0t = w0t_ref[...]
    r = (x0 * w0t[0:1, :] + x1 * w0t[1:2, :] + x2 * w0t[2:3, :])
    pts = ptsq_ref[0]
    v = jnp.dot(pts, w0t[3:67, :], preferred_element_type=jnp.float32,
                precision=lax.Precision.HIGHEST)
    v_ref[0] = v + r + b0_ref[...]
    r_ref[0] = r


def _knn_call(xyzt, xyz, points, W0T, b0r):
    B, N, _ = xyz.shape
    grid = (B, N // NQ)
    return pl.pallas_call(
        _knn_body,
        grid=grid,
        in_specs=[
            pl.BlockSpec((1, 3, N), lambda b, q: (b, 0, 0)),
            pl.BlockSpec((1, NQ, 3), lambda b, q: (b, q, 0)),
            pl.BlockSpec((1, NQ, 64), lambda b, q: (b, q, 0)),
            pl.BlockSpec((67, 64), lambda b, q: (0, 0)),
            pl.BlockSpec((1, 64), lambda b, q: (0, 0)),
        ],
        out_specs=[
            pl.BlockSpec((1, NQ, KNN), lambda b, q: (b, q, 0)),
            pl.BlockSpec((1, NQ, 64), lambda b, q: (b, q, 0)),
            pl.BlockSpec((1, NQ, 64), lambda b, q: (b, q, 0)),
        ],
        out_shape=[
            jax.ShapeDtypeStruct((B, N, KNN), jnp.int32),
            jax.ShapeDtypeStruct((B, N, 64), jnp.float32),
            jax.ShapeDtypeStruct((B, N, 64), jnp.float32),
        ],
    )(xyzt, xyz, points, W0T, b0r)


def _sc_gather(table, idx3):
    NW = 32
    NCHUNK = idx3.shape[1]
    mesh = plsc.VectorSubcoreMesh(core_axis_name="c", subcore_axis_name="s")

    @functools.partial(
        pl.kernel,
        mesh=mesh,
        compiler_params=pltpu.CompilerParams(use_tc_tiling_on_sc=False),
        out_type=jax.ShapeDtypeStruct((NW * NCHUNK, 128, 64), jnp.float32),
        scratch_types=[
            pltpu.VMEM((NCHUNK, 128), jnp.int32),
            pltpu.VMEM((128, 64), jnp.float32),
            pltpu.SemaphoreType.DMA,
        ],
    )
    def k(table_hbm, idx_hbm, out_hbm, idx_v, rows_v, sem):
        w = lax.axis_index("s") * 2 + lax.axis_index("c")
        pltpu.sync_copy(idx_hbm.at[w], idx_v)

        def body(j, carry):
            pltpu.async_copy(table_hbm.at[idx_v.at[j]], rows_v, sem).wait()
            pltpu.sync_copy(rows_v, out_hbm.at[w * NCHUNK + j])
            return carry

        lax.fori_loop(0, NCHUNK, body, 0)

    return k(table, idx3)


def _stats_body(raw_ref, r_ref, out_ref):
    g = pl.program_id(0)
    raw = raw_ref[...]
    r = r_ref[...]
    x1 = raw - jnp.repeat(r, KNN, axis=0)
    s = jnp.sum(x1, axis=0, keepdims=True)
    s2 = jnp.sum(x1 * x1, axis=0, keepdims=True)

    @pl.when(g == 0)
    def _():
        out_ref[...] = jnp.zeros_like(out_ref)

    out_ref[0:1, :] += s
    out_ref[1:2, :] += s2


def _stats_call(raw, R):
    BN = R.shape[0]
    grid = (BN // QB,)
    return pl.pallas_call(
        _stats_body,
        grid=grid,
        in_specs=[
            pl.BlockSpec((QB * KNN, 64), lambda g: (g, 0)),
            pl.BlockSpec((QB, 64), lambda g: (g, 0)),
        ],
        out_specs=pl.BlockSpec((8, 64), lambda g: (0, 0)),
        out_shape=jax.ShapeDtypeStruct((8, 64), jnp.float32),
    )(raw, R)


def _layer_body(raw_ref, r_ref, sc_ref, sh_ref, wt_ref, x2_ref, st_ref):
    g = pl.program_id(0)
    raw = raw_ref[...]
    r = r_ref[...]
    if r.shape[0] == QB:
        x = raw - jnp.repeat(r, KNN, axis=0)
    else:
        x = raw
    h = jnp.maximum(x * sc_ref[...] + sh_ref[...], 0.0)
    x2 = jnp.dot(h, wt_ref[...], preferred_element_type=jnp.float32,
                 precision=lax.Precision.HIGHEST)
    x2_ref[...] = x2
    s = jnp.sum(x2, axis=0, keepdims=True)
    s2 = jnp.sum(x2 * x2, axis=0, keepdims=True)

    @pl.when(g == 0)
    def _():
        st_ref[...] = jnp.zeros_like(st_ref)

    st_ref[0:1, :] += s
    st_ref[1:2, :] += s2


def _layer_call(raw, R, scale, shift, WT):
    M, Cin = raw.shape
    Cout = WT.shape[1]
    grid = (M // (QB * KNN),)
    use_r = R is not None
    rr = R if use_r else jnp.zeros((1, 1), jnp.float32)
    rspec = (pl.BlockSpec((QB, Cin), lambda g: (g, 0)) if use_r
             else pl.BlockSpec((1, 1), lambda g: (0, 0)))
    return pl.pallas_call(
        _layer_body,
        grid=grid,
        in_specs=[
            pl.BlockSpec((QB * KNN, Cin), lambda g: (g, 0)),
            rspec,
            pl.BlockSpec((1, Cin), lambda g: (0, 0)),
            pl.BlockSpec((1, Cin), lambda g: (0, 0)),
            pl.BlockSpec((Cin, Cout), lambda g: (0, 0)),
        ],
        out_specs=[
            pl.BlockSpec((QB * KNN, Cout), lambda g: (g, 0)),
            pl.BlockSpec((8, Cout), lambda g: (0, 0)),
        ],
        out_shape=[
            jax.ShapeDtypeStruct((M, Cout), jnp.float32),
            jax.ShapeDtypeStruct((8, Cout), jnp.float32),
        ],
    )(raw, rr, scale, shift, WT)


def _final_body(x3_ref, sc_ref, sh_ref, out_ref):
    x3 = x3_ref[...]
    y = jnp.maximum(x3 * sc_ref[...] + sh_ref[...], 0.0)
    y = y.reshape(QB, KNN, 128)
    out_ref[...] = jnp.max(y, axis=1)


def _final_call(x3, scale, shift):
    M = x3.shape[0]
    BN = M // KNN
    grid = (BN // QB,)
    return pl.pallas_call(
        _final_body,
        grid=grid,
        in_specs=[
            pl.BlockSpec((QB * KNN, 128), lambda g: (g, 0)),
            pl.BlockSpec((1, 128), lambda g: (0, 0)),
            pl.BlockSpec((1, 128), lambda g: (0, 0)),
        ],
        out_specs=pl.BlockSpec((QB, 128), lambda g: (g, 0)),
        out_shape=jax.ShapeDtypeStruct((BN, 128), jnp.float32),
    )(x3, scale, shift)


def _finalize(stats, cnt, gm, bt):
    s = stats[0, :gm.shape[0]]
    s2 = stats[1, :gm.shape[0]]
    mu = s / cnt
    var = s2 / cnt - mu * mu
    scale = gm / jnp.sqrt(var + EPS)
    shift = bt - mu * scale
    return scale[None, :], shift[None, :]


def kernel(xyz, points, W0, b0, g0, beta0, W1, b1, g1, beta1, W2, b2, g2, beta2):
    B, N, _ = xyz.shape
    BN = B * N
    cnt = jnp.float32(BN * KNN)

    xyzt = jnp.swapaxes(xyz, 1, 2)
    W0T = jnp.swapaxes(W0, 0, 1)
    idx, V, R = _knn_call(xyzt, xyz, points, W0T, b0[None, :])

    idx_flat = idx.reshape(-1)
    NW = 32
    NCHUNK = (BN * KNN) // (NW * 128)
    idx3 = idx_flat.reshape(NW, NCHUNK, 128)
    raw = _sc_gather(V.reshape(BN, 64), idx3).reshape(BN * KNN, 64)

    Rf = R.reshape(BN, 64)
    st1 = _stats_call(raw, Rf)
    sc1, sh1 = _finalize(st1, cnt, g0, beta0)

    W1T = jnp.swapaxes(W1, 0, 1)
    x2, st2 = _layer_call(raw, Rf, sc1, sh1, W1T)
    sc2, sh2 = _finalize(st2, cnt, g1, beta1)

    W2T = jnp.swapaxes(W2, 0, 1)
    x3, st3 = _layer_call(x2, None, sc2, sh2, W2T)
    sc3, sh3 = _finalize(st3, cnt, g2, beta2)

    out = _final_call(x3, sc3, sh3)
    return out.reshape(B, N, 128)


# speedup vs baseline: 5.5724x; 1.0173x over previous
import functools

import jax
import jax.numpy as jnp
from jax import lax
from jax.experimental import pallas as pl
from jax.experimental.pallas import tpu as pltpu
from jax.experimental.pallas import tpu_sc as plsc

KNN = 32
NQ = 512
QB = 128
EPS = 1e-5


def _knn_body(xyzt_ref, xyzq_ref, ptsq_ref, w0t_ref, b0_ref, idx_ref, v_ref, r_ref):
    b = pl.program_id(0)
    N = xyzt_ref.shape[2]
    xt = xyzt_ref[0]
    q = xyzq_ref[0]
    X0, X1, X2 = xt[0:1, :], xt[1:2, :], xt[2:3, :]
    x0, x1, x2 = q[:, 0:1], q[:, 1:2], q[:, 2:3]
    sqm = X0 * X0 + X1 * X1 + X2 * X2
    sqn = x0 * x0 + x1 * x1 + x2 * x2
    dot = lax.dot_general(q, xt, (((1,), (0,)), ((), ())),
                          preferred_element_type=jnp.float32)
    S = (sqn + sqm) - 2.0 * dot

    col = lax.broadcasted_iota(jnp.int32, (NQ, N), 1)
    kcol = lax.broadcasted_iota(jnp.int32, (NQ, KNN), 1)
    BIG = jnp.float32(jnp.finfo(jnp.float32).max)

    def body(i, carry):
        s, ids = carry
        am = jnp.argmin(s, axis=1).astype(jnp.int32)[:, None]
        ids = jnp.where(kcol == i, am, ids)
        s = jnp.where(col == am, BIG, s)
        return s, ids

    ids0 = jnp.full((NQ, KNN), 0, jnp.int32)
    _, ids = lax.fori_loop(0, KNN, body, (S, ids0))
    idx_ref[0] = ids + b * N

    w0t = w0t_ref[...]
    r = (x0 * w0t[0:1, :] + x1 * w0t[1:2, :] + x2 * w0t[2:3, :])
    pts = ptsq_ref[0]
    v = jnp.dot(pts, w0t[3:67, :], preferred_element_type=jnp.float32,
                precision=lax.Precision.HIGHEST)
    v_ref[0] = v + r + b0_ref[...]
    r_ref[0] = r


def _knn_call(xyzt, xyz, points, W0T, b0r):
    B, N, _ = xyz.shape
    grid = (B, N // NQ)
    return pl.pallas_call(
        _knn_body,
        grid=grid,
        in_specs=[
            pl.BlockSpec((1, 3, N), lambda b, q: (b, 0, 0)),
            pl.BlockSpec((1, NQ, 3), lambda b, q: (b, q, 0)),
            pl.BlockSpec((1, NQ, 64), lambda b, q: (b, q, 0)),
            pl.BlockSpec((67, 64), lambda b, q: (0, 0)),
            pl.BlockSpec((1, 64), lambda b, q: (0, 0)),
        ],
        out_specs=[
            pl.BlockSpec((1, NQ, KNN), lambda b, q: (b, q, 0)),
            pl.BlockSpec((1, NQ, 64), lambda b, q: (b, q, 0)),
            pl.BlockSpec((1, NQ, 64), lambda b, q: (b, q, 0)),
        ],
        out_shape=[
            jax.ShapeDtypeStruct((B, N, KNN), jnp.int32),
            jax.ShapeDtypeStruct((B, N, 64), jnp.float32),
            jax.ShapeDtypeStruct((B, N, 64), jnp.float32),
        ],
    )(xyzt, xyz, points, W0T, b0r)


def _sc_gather(table, idx3):
    NW = 32
    NCHUNK = idx3.shape[1]
    mesh = plsc.VectorSubcoreMesh(core_axis_name="c", subcore_axis_name="s")

    @functools.partial(
        pl.kernel,
        mesh=mesh,
        compiler_params=pltpu.CompilerParams(use_tc_tiling_on_sc=False),
        out_type=jax.ShapeDtypeStruct((NW * NCHUNK, 128, 64), jnp.float32),
        scratch_types=[
            pltpu.VMEM((NCHUNK, 128), jnp.int32),
            pltpu.VMEM((128, 64), jnp.float32),
            pltpu.SemaphoreType.DMA,
        ],
    )
    def k(table_hbm, idx_hbm, out_hbm, idx_v, rows_v, sem):
        w = lax.axis_index("s") * 2 + lax.axis_index("c")
        pltpu.sync_copy(idx_hbm.at[w], idx_v)

        def body(j, carry):
            pltpu.async_copy(table_hbm.at[idx_v.at[j]], rows_v, sem).wait()
            pltpu.sync_copy(rows_v, out_hbm.at[w * NCHUNK + j])
            return carry

        lax.fori_loop(0, NCHUNK, body, 0)

    return k(table, idx3)


def _stats_body(raw_ref, r_ref, out_ref):
    g = pl.program_id(0)
    raw = raw_ref[...]
    r = r_ref[...]
    x1 = raw - jnp.repeat(r, KNN, axis=0)
    s = jnp.sum(x1, axis=0, keepdims=True)
    s2 = jnp.sum(x1 * x1, axis=0, keepdims=True)

    @pl.when(g == 0)
    def _():
        out_ref[...] = jnp.zeros_like(out_ref)

    out_ref[0:1, :] += s
    out_ref[1:2, :] += s2


def _stats_call(raw, R):
    BN = R.shape[0]
    grid = (BN // QB,)
    return pl.pallas_call(
        _stats_body,
        grid=grid,
        in_specs=[
            pl.BlockSpec((QB * KNN, 64), lambda g: (g, 0)),
            pl.BlockSpec((QB, 64), lambda g: (g, 0)),
        ],
        out_specs=pl.BlockSpec((8, 64), lambda g: (0, 0)),
        out_shape=jax.ShapeDtypeStruct((8, 64), jnp.float32),
    )(raw, R)


def _layer_body(raw_ref, r_ref, sc_ref, sh_ref, wt_ref, x2_ref, st_ref):
    g = pl.program_id(0)
    raw = raw_ref[...]
    r = r_ref[...]
    if r.shape[0] == QB:
        x = raw - jnp.repeat(r, KNN, axis=0)
    else:
        x = raw
    h = jnp.maximum(x * sc_ref[...] + sh_ref[...], 0.0)
    x2 = jnp.dot(h, wt_ref[...], preferred_element_type=jnp.float32,
                 precision=lax.Precision.HIGHEST)
    x2_ref[...] = x2
    s = jnp.sum(x2, axis=0, keepdims=True)
    s2 = jnp.sum(x2 * x2, axis=0, keepdims=True)

    @pl.when(g == 0)
    def _():
        st_ref[...] = jnp.zeros_like(st_ref)

    st_ref[0:1, :] += s
    st_ref[1:2, :] += s2


def _layer_call(raw, R, scale, shift, WT):
    M, Cin = raw.shape
    Cout = WT.shape[1]
    grid = (M // (QB * KNN),)
    use_r = R is not None
    rr = R if use_r else jnp.zeros((1, 1), jnp.float32)
    rspec = (pl.BlockSpec((QB, Cin), lambda g: (g, 0)) if use_r
             else pl.BlockSpec((1, 1), lambda g: (0, 0)))
    return pl.pallas_call(
        _layer_body,
        grid=grid,
        in_specs=[
            pl.BlockSpec((QB * KNN, Cin), lambda g: (g, 0)),
            rspec,
            pl.BlockSpec((1, Cin), lambda g: (0, 0)),
            pl.BlockSpec((1, Cin), lambda g: (0, 0)),
            pl.BlockSpec((Cin, Cout), lambda g: (0, 0)),
        ],
        out_specs=[
            pl.BlockSpec((QB * KNN, Cout), lambda g: (g, 0)),
            pl.BlockSpec((8, Cout), lambda g: (0, 0)),
        ],
        out_shape=[
            jax.ShapeDtypeStruct((M, Cout), jnp.float32),
            jax.ShapeDtypeStruct((8, Cout), jnp.float32),
        ],
    )(raw, rr, scale, shift, WT)


def _final_body(x3_ref, sc_ref, sh_ref, out_ref):
    x3 = x3_ref[...]
    y = jnp.maximum(x3 * sc_ref[...] + sh_ref[...], 0.0)
    y = y.reshape(QB, KNN, 128)
    out_ref[...] = jnp.max(y, axis=1)


def _final_call(x3, scale, shift):
    M = x3.shape[0]
    BN = M // KNN
    grid = (BN // QB,)
    return pl.pallas_call(
        _final_body,
        grid=grid,
        in_specs=[
            pl.BlockSpec((QB * KNN, 128), lambda g: (g, 0)),
            pl.BlockSpec((1, 128), lambda g: (0, 0)),
            pl.BlockSpec((1, 128), lambda g: (0, 0)),
        ],
        out_specs=pl.BlockSpec((QB, 128), lambda g: (g, 0)),
        out_shape=jax.ShapeDtypeStruct((BN, 128), jnp.float32),
    )(x3, scale, shift)


def _finalize(stats, cnt, gm, bt):
    s = stats[0, :gm.shape[0]]
    s2 = stats[1, :gm.shape[0]]
    mu = s / cnt
    var = s2 / cnt - mu * mu
    scale = gm / jnp.sqrt(var + EPS)
    shift = bt - mu * scale
    return scale[None, :], shift[None, :]


def kernel(xyz, points, W0, b0, g0, beta0, W1, b1, g1, beta1, W2, b2, g2, beta2):
    B, N, _ = xyz.shape
    BN = B * N
    cnt = jnp.float32(BN * KNN)

    xyzt = jnp.swapaxes(xyz, 1, 2)
    W0T = jnp.swapaxes(W0, 0, 1)
    idx, V, R = _knn_call(xyzt, xyz, points, W0T, b0[None, :])

    idx_flat = idx.reshape(-1)
    NW = 32
    NCHUNK = (BN * KNN) // (NW * 128)
    idx3 = idx_flat.reshape(NW, NCHUNK, 128)
    raw = _sc_gather(V.reshape(BN, 64), idx3).reshape(BN * KNN, 64)

    Rf = R.reshape(BN, 64)
    st1 = _stats_call(raw, Rf)
    sc1, sh1 = _finalize(st1, cnt, g0, beta0)

    W1T = jnp.swapaxes(W1, 0, 1)
    x2, st2 = _layer_call(raw, Rf, sc1, sh1, W1T)
    sc2, sh2 = _finalize(st2, cnt, g1, beta1)

    W2T = jnp.swapaxes(W2, 0, 1)
    x3, st3 = _layer_call(x2, None, sc2, sh2, W2T)
    sc3, sh3 = _finalize(st3, cnt, g2, beta2)

    out = _final_call(x3, sc3, sh3)
    return out.reshape(B, N, 128)


# speedup vs baseline: 6.2359x; 1.1191x over previous
import functools

import jax
import jax.numpy as jnp
from jax import lax
from jax.experimental import pallas as pl
from jax.experimental.pallas import tpu as pltpu
from jax.experimental.pallas import tpu_sc as plsc

KNN = 32
NQ = 512
QB = 128
EPS = 1e-5


def _knn_body(xyzt_ref, xyzq_ref, ptsq_ref, w0t_ref, b0_ref, idx_ref, v_ref, r_ref):
    b = pl.program_id(0)
    N = xyzt_ref.shape[2]
    xt = xyzt_ref[0]
    q = xyzq_ref[0]
    X0, X1, X2 = xt[0:1, :], xt[1:2, :], xt[2:3, :]
    x0, x1, x2 = q[:, 0:1], q[:, 1:2], q[:, 2:3]
    sqm = X0 * X0 + X1 * X1 + X2 * X2
    sqn = x0 * x0 + x1 * x1 + x2 * x2
    dot = lax.dot_general(q, xt, (((1,), (0,)), ((), ())),
                          preferred_element_type=jnp.float32)
    S = (sqn + sqm) - 2.0 * dot

    col = lax.broadcasted_iota(jnp.int32, (NQ, N), 1)
    kcol = lax.broadcasted_iota(jnp.int32, (NQ, KNN), 1)
    BIG = jnp.float32(jnp.finfo(jnp.float32).max)

    def body(i, carry):
        s, ids = carry
        am1 = jnp.argmin(s, axis=1).astype(jnp.int32)[:, None]
        ids = jnp.where(kcol == 2 * i, am1, ids)
        m1 = col == am1
        am2 = jnp.argmin(jnp.where(m1, BIG, s), axis=1).astype(jnp.int32)[:, None]
        ids = jnp.where(kcol == 2 * i + 1, am2, ids)
        s = jnp.where(m1 | (col == am2), BIG, s)
        return s, ids

    ids0 = jnp.full((NQ, KNN), 0, jnp.int32)
    _, ids = lax.fori_loop(0, KNN // 2, body, (S, ids0))
    idx_ref[0] = ids + b * N

    w0t = w0t_ref[...]
    r = (x0 * w0t[0:1, :] + x1 * w0t[1:2, :] + x2 * w0t[2:3, :])
    pts = ptsq_ref[0]
    v = jnp.dot(pts, w0t[3:67, :], preferred_element_type=jnp.float32,
                precision=lax.Precision.HIGHEST)
    v_ref[0] = v + r + b0_ref[...]
    r_ref[0] = r


def _knn_call(xyzt, xyz, points, W0T, b0r):
    B, N, _ = xyz.shape
    grid = (B, N // NQ)
    return pl.pallas_call(
        _knn_body,
        grid=grid,
        in_specs=[
            pl.BlockSpec((1, 3, N), lambda b, q: (b, 0, 0)),
            pl.BlockSpec((1, NQ, 3), lambda b, q: (b, q, 0)),
            pl.BlockSpec((1, NQ, 64), lambda b, q: (b, q, 0)),
            pl.BlockSpec((67, 64), lambda b, q: (0, 0)),
            pl.BlockSpec((1, 64), lambda b, q: (0, 0)),
        ],
        out_specs=[
            pl.BlockSpec((1, NQ, KNN), lambda b, q: (b, q, 0)),
            pl.BlockSpec((1, NQ, 64), lambda b, q: (b, q, 0)),
            pl.BlockSpec((1, NQ, 64), lambda b, q: (b, q, 0)),
        ],
        out_shape=[
            jax.ShapeDtypeStruct((B, N, KNN), jnp.int32),
            jax.ShapeDtypeStruct((B, N, 64), jnp.float32),
            jax.ShapeDtypeStruct((B, N, 64), jnp.float32),
        ],
    )(xyzt, xyz, points, W0T, b0r)


def _sc_gather(table, idx3):
    NW = 32
    NCHUNK = idx3.shape[1]
    mesh = plsc.VectorSubcoreMesh(core_axis_name="c", subcore_axis_name="s")

    @functools.partial(
        pl.kernel,
        mesh=mesh,
        compiler_params=pltpu.CompilerParams(use_tc_tiling_on_sc=False),
        out_type=jax.ShapeDtypeStruct((NW * NCHUNK, 128, 64), jnp.float32),
        scratch_types=[
            pltpu.VMEM((NCHUNK, 128), jnp.int32),
            pltpu.VMEM((128, 64), jnp.float32),
            pltpu.SemaphoreType.DMA,
        ],
    )
    def k(table_hbm, idx_hbm, out_hbm, idx_v, rows_v, sem):
        w = lax.axis_index("s") * 2 + lax.axis_index("c")
        pltpu.sync_copy(idx_hbm.at[w], idx_v)

        def body(j, carry):
            pltpu.async_copy(table_hbm.at[idx_v.at[j]], rows_v, sem).wait()
            pltpu.sync_copy(rows_v, out_hbm.at[w * NCHUNK + j])
            return carry

        lax.fori_loop(0, NCHUNK, body, 0)

    return k(table, idx3)


def _stats_body(raw_ref, r_ref, out_ref):
    g = pl.program_id(0)
    raw = raw_ref[...]
    r = r_ref[...]
    x1 = raw - jnp.repeat(r, KNN, axis=0)
    s = jnp.sum(x1, axis=0, keepdims=True)
    s2 = jnp.sum(x1 * x1, axis=0, keepdims=True)

    @pl.when(g == 0)
    def _():
        out_ref[...] = jnp.zeros_like(out_ref)

    out_ref[0:1, :] += s
    out_ref[1:2, :] += s2


def _stats_call(raw, R):
    BN = R.shape[0]
    grid = (BN // QB,)
    return pl.pallas_call(
        _stats_body,
        grid=grid,
        in_specs=[
            pl.BlockSpec((QB * KNN, 64), lambda g: (g, 0)),
            pl.BlockSpec((QB, 64), lambda g: (g, 0)),
        ],
        out_specs=pl.BlockSpec((8, 64), lambda g: (0, 0)),
        out_shape=jax.ShapeDtypeStruct((8, 64), jnp.float32),
    )(raw, R)


def _layer_body(raw_ref, r_ref, sc_ref, sh_ref, wt_ref, x2_ref, st_ref):
    g = pl.program_id(0)
    raw = raw_ref[...]
    r = r_ref[...]
    if r.shape[0] == QB:
        x = raw - jnp.repeat(r, KNN, axis=0)
    else:
        x = raw
    h = jnp.maximum(x * sc_ref[...] + sh_ref[...], 0.0)
    x2 = jnp.dot(h, wt_ref[...], preferred_element_type=jnp.float32,
                 precision=lax.Precision.HIGHEST)
    x2_ref[...] = x2
    s = jnp.sum(x2, axis=0, keepdims=True)
    s2 = jnp.sum(x2 * x2, axis=0, keepdims=True)

    @pl.when(g == 0)
    def _():
        st_ref[...] = jnp.zeros_like(st_ref)

    st_ref[0:1, :] += s
    st_ref[1:2, :] += s2


def _layer_call(raw, R, scale, shift, WT):
    M, Cin = raw.shape
    Cout = WT.shape[1]
    grid = (M // (QB * KNN),)
    use_r = R is not None
    rr = R if use_r else jnp.zeros((1, 1), jnp.float32)
    rspec = (pl.BlockSpec((QB, Cin), lambda g: (g, 0)) if use_r
             else pl.BlockSpec((1, 1), lambda g: (0, 0)))
    return pl.pallas_call(
        _layer_body,
        grid=grid,
        in_specs=[
            pl.BlockSpec((QB * KNN, Cin), lambda g: (g, 0)),
            rspec,
            pl.BlockSpec((1, Cin), lambda g: (0, 0)),
            pl.BlockSpec((1, Cin), lambda g: (0, 0)),
            pl.BlockSpec((Cin, Cout), lambda g: (0, 0)),
        ],
        out_specs=[
            pl.BlockSpec((QB * KNN, Cout), lambda g: (g, 0)),
            pl.BlockSpec((8, Cout), lambda g: (0, 0)),
        ],
        out_shape=[
            jax.ShapeDtypeStruct((M, Cout), jnp.float32),
            jax.ShapeDtypeStruct((8, Cout), jnp.float32),
        ],
    )(raw, rr, scale, shift, WT)


def _final_body(x3_ref, sc_ref, sh_ref, out_ref):
    x3 = x3_ref[...]
    y = jnp.maximum(x3 * sc_ref[...] + sh_ref[...], 0.0)
    y = y.reshape(QB, KNN, 128)
    out_ref[...] = jnp.max(y, axis=1)


def _final_call(x3, scale, shift):
    M = x3.shape[0]
    BN = M // KNN
    grid = (BN // QB,)
    return pl.pallas_call(
        _final_body,
        grid=grid,
        in_specs=[
            pl.BlockSpec((QB * KNN, 128), lambda g: (g, 0)),
            pl.BlockSpec((1, 128), lambda g: (0, 0)),
            pl.BlockSpec((1, 128), lambda g: (0, 0)),
        ],
        out_specs=pl.BlockSpec((QB, 128), lambda g: (g, 0)),
        out_shape=jax.ShapeDtypeStruct((BN, 128), jnp.float32),
    )(x3, scale, shift)


def _finalize(stats, cnt, gm, bt):
    s = stats[0, :gm.shape[0]]
    s2 = stats[1, :gm.shape[0]]
    mu = s / cnt
    var = s2 / cnt - mu * mu
    scale = gm / jnp.sqrt(var + EPS)
    shift = bt - mu * scale
    return scale[None, :], shift[None, :]


def kernel(xyz, points, W0, b0, g0, beta0, W1, b1, g1, beta1, W2, b2, g2, beta2):
    B, N, _ = xyz.shape
    BN = B * N
    cnt = jnp.float32(BN * KNN)

    xyzt = jnp.swapaxes(xyz, 1, 2)
    W0T = jnp.swapaxes(W0, 0, 1)
    idx, V, R = _knn_call(xyzt, xyz, points, W0T, b0[None, :])

    idx_flat = idx.reshape(-1)
    NW = 32
    NCHUNK = (BN * KNN) // (NW * 128)
    idx3 = idx_flat.reshape(NW, NCHUNK, 128)
    raw = _sc_gather(V.reshape(BN, 64), idx3).reshape(BN * KNN, 64)

    Rf = R.reshape(BN, 64)
    st1 = _stats_call(raw, Rf)
    sc1, sh1 = _finalize(st1, cnt, g0, beta0)

    W1T = jnp.swapaxes(W1, 0, 1)
    x2, st2 = _layer_call(raw, Rf, sc1, sh1, W1T)
    sc2, sh2 = _finalize(st2, cnt, g1, beta1)

    W2T = jnp.swapaxes(W2, 0, 1)
    x3, st3 = _layer_call(x2, None, sc2, sh2, W2T)
    sc3, sh3 = _finalize(st3, cnt, g2, beta2)

    out = _final_call(x3, sc3, sh3)
    return out.reshape(B, N, 128)
